# baseline (speedup 1.0000x reference)
import functools

import jax
import jax.numpy as jnp
from jax import lax
from jax.experimental import pallas as pl
from jax.experimental.pallas import tpu as pltpu
from jax.experimental.pallas import tpu_sc as plsc

NUM_EMB = 1024
DIM = 768
BATCH = 64

_NC = 2
_NS = 16
_NW = _NC * _NS
_ROWS_PER_W = NUM_EMB // _NW


def _sc_gather(pos_table, emb_indices):
    mesh = plsc.VectorSubcoreMesh(core_axis_name="c", subcore_axis_name="s")

    @functools.partial(
        pl.kernel,
        mesh=mesh,
        out_type=jax.ShapeDtypeStruct((NUM_EMB, DIM), jnp.float32),
        scratch_types=[
            pltpu.VMEM((_ROWS_PER_W,), jnp.int32),
            pltpu.VMEM((_ROWS_PER_W, DIM), jnp.float32),
            pltpu.SemaphoreType.DMA,
        ],
    )
    def gather_kernel(table_hbm, idx_hbm, out_hbm, idx_v, rows_v, sem):
        wid = lax.axis_index("s") * _NC + lax.axis_index("c")
        base = wid * _ROWS_PER_W
        pltpu.sync_copy(idx_hbm.at[pl.ds(base, _ROWS_PER_W)], idx_v)
        pltpu.async_copy(table_hbm.at[idx_v], rows_v, sem).wait()
        pltpu.sync_copy(rows_v, out_hbm.at[pl.ds(base, _ROWS_PER_W)])

    return gather_kernel(pos_table, emb_indices)


def _add_body(pos_ref, x_ref, o_ref):
    o_ref[...] = x_ref[...] + pos_ref[...]


def _tc_add(x, pos):
    return pl.pallas_call(
        _add_body,
        grid=(BATCH,),
        in_specs=[
            pl.BlockSpec((NUM_EMB, DIM), lambda b: (0, 0)),
            pl.BlockSpec((1, NUM_EMB, DIM), lambda b: (b, 0, 0)),
        ],
        out_specs=pl.BlockSpec((1, NUM_EMB, DIM), lambda b: (b, 0, 0)),
        out_shape=jax.ShapeDtypeStruct((BATCH, NUM_EMB, DIM), jnp.float32),
        compiler_params=pltpu.CompilerParams(
            dimension_semantics=("arbitrary",),
        ),
    )(pos, x)


def kernel(x, pos_table, emb_indices):
    pos = _sc_gather(pos_table, emb_indices)
    return _tc_add(x, pos)


# speedup vs baseline: 1.0288x; 1.0288x over previous
import functools

import jax
import jax.numpy as jnp
from jax import lax
from jax.experimental import pallas as pl
from jax.experimental.pallas import tpu as pltpu
from jax.experimental.pallas import tpu_sc as plsc

NUM_EMB = 1024
DIM = 768
BATCH = 64

_NC = 2
_NS = 16
_NW = _NC * _NS
_ROWS_PER_W = NUM_EMB // _NW


def _sc_gather(pos_table, emb_indices):
    mesh = plsc.VectorSubcoreMesh(core_axis_name="c", subcore_axis_name="s")

    @functools.partial(
        pl.kernel,
        mesh=mesh,
        out_type=jax.ShapeDtypeStruct((NUM_EMB, DIM), jnp.float32),
        scratch_types=[
            pltpu.VMEM((_ROWS_PER_W,), jnp.int32),
            pltpu.VMEM((_ROWS_PER_W, DIM), jnp.float32),
            pltpu.SemaphoreType.DMA,
        ],
    )
    def gather_kernel(table_hbm, idx_hbm, out_hbm, idx_v, rows_v, sem):
        wid = lax.axis_index("s") * _NC + lax.axis_index("c")
        base = wid * _ROWS_PER_W
        pltpu.sync_copy(idx_hbm.at[pl.ds(base, _ROWS_PER_W)], idx_v)
        pltpu.async_copy(table_hbm.at[idx_v], rows_v, sem).wait()
        pltpu.sync_copy(rows_v, out_hbm.at[pl.ds(base, _ROWS_PER_W)])

    return gather_kernel(pos_table, emb_indices)


_BB = 2


def _add_body(pos_ref, x_ref, o_ref):
    o_ref[...] = x_ref[...] + pos_ref[...][None]


def _tc_add(x, pos):
    return pl.pallas_call(
        _add_body,
        grid=(BATCH // _BB,),
        in_specs=[
            pl.BlockSpec((NUM_EMB, DIM), lambda b: (0, 0)),
            pl.BlockSpec((_BB, NUM_EMB, DIM), lambda b: (b, 0, 0)),
        ],
        out_specs=pl.BlockSpec((_BB, NUM_EMB, DIM), lambda b: (b, 0, 0)),
        out_shape=jax.ShapeDtypeStruct((BATCH, NUM_EMB, DIM), jnp.float32),
        compiler_params=pltpu.CompilerParams(
            dimension_semantics=("arbitrary",),
        ),
    )(pos, x)


def kernel(x, pos_table, emb_indices):
    pos = _sc_gather(pos_table, emb_indices)
    return _tc_add(x, pos)


# speedup vs baseline: 1.0378x; 1.0088x over previous
import functools

import jax
import jax.numpy as jnp
from jax import lax
from jax.experimental import pallas as pl
from jax.experimental.pallas import tpu as pltpu
from jax.experimental.pallas import tpu_sc as plsc

NUM_EMB = 1024
DIM = 768
BATCH = 64

_NC = 2
_NS = 16
_NW = _NC * _NS
_ROWS_PER_W = NUM_EMB // _NW


def _sc_gather(pos_table, emb_indices):
    mesh = plsc.VectorSubcoreMesh(core_axis_name="c", subcore_axis_name="s")

    @functools.partial(
        pl.kernel,
        mesh=mesh,
        out_type=jax.ShapeDtypeStruct((NUM_EMB, DIM), jnp.float32),
        scratch_types=[
            pltpu.VMEM((_ROWS_PER_W,), jnp.int32),
            pltpu.VMEM((_ROWS_PER_W, DIM), jnp.float32),
            pltpu.SemaphoreType.DMA,
        ],
    )
    def gather_kernel(table_hbm, idx_hbm, out_hbm, idx_v, rows_v, sem):
        wid = lax.axis_index("s") * _NC + lax.axis_index("c")
        base = wid * _ROWS_PER_W
        pltpu.sync_copy(idx_hbm.at[pl.ds(base, _ROWS_PER_W)], idx_v)
        pltpu.async_copy(table_hbm.at[idx_v], rows_v, sem).wait()
        pltpu.sync_copy(rows_v, out_hbm.at[pl.ds(base, _ROWS_PER_W)])

    return gather_kernel(pos_table, emb_indices)


_BB = 4


def _add_body(pos_ref, x_ref, o_ref):
    o_ref[...] = x_ref[...] + pos_ref[...][None]


def _tc_add(x, pos):
    return pl.pallas_call(
        _add_body,
        grid=(BATCH // _BB,),
        in_specs=[
            pl.BlockSpec((NUM_EMB, DIM), lambda b: (0, 0)),
            pl.BlockSpec((_BB, NUM_EMB, DIM), lambda b: (b, 0, 0)),
        ],
        out_specs=pl.BlockSpec((_BB, NUM_EMB, DIM), lambda b: (b, 0, 0)),
        out_shape=jax.ShapeDtypeStruct((BATCH, NUM_EMB, DIM), jnp.float32),
        compiler_params=pltpu.CompilerParams(
            dimension_semantics=("arbitrary",),
        ),
    )(pos, x)


def kernel(x, pos_table, emb_indices):
    pos = _sc_gather(pos_table, emb_indices)
    return _tc_add(x, pos)


# speedup vs baseline: 1.2246x; 1.1800x over previous
import functools

import jax
import jax.numpy as jnp
from jax import lax
from jax.experimental import pallas as pl
from jax.experimental.pallas import tpu as pltpu
from jax.experimental.pallas import tpu_sc as plsc

NUM_EMB = 1024
DIM = 768
BATCH = 64

_NC = 2
_NS = 16
_NW = _NC * _NS
_ROWS_PER_W = NUM_EMB // _NW


def _sc_gather(pos_table, emb_indices):
    mesh = plsc.VectorSubcoreMesh(core_axis_name="c", subcore_axis_name="s")

    @functools.partial(
        pl.kernel,
        mesh=mesh,
        out_type=jax.ShapeDtypeStruct((NUM_EMB, DIM), jnp.float32),
        scratch_types=[
            pltpu.VMEM((_ROWS_PER_W,), jnp.int32),
            pltpu.VMEM((_ROWS_PER_W, DIM), jnp.float32),
            pltpu.SemaphoreType.DMA,
        ],
    )
    def gather_kernel(table_hbm, idx_hbm, out_hbm, idx_v, rows_v, sem):
        wid = lax.axis_index("s") * _NC + lax.axis_index("c")
        base = wid * _ROWS_PER_W
        pltpu.sync_copy(idx_hbm.at[pl.ds(base, _ROWS_PER_W)], idx_v)
        pltpu.async_copy(table_hbm.at[idx_v], rows_v, sem).wait()
        pltpu.sync_copy(rows_v, out_hbm.at[pl.ds(base, _ROWS_PER_W)])

    return gather_kernel(pos_table, emb_indices)


_BB = 4


def _add_body(pos_ref, x_ref, o_ref):
    o_ref[...] = x_ref[...] + pos_ref[...][None]


def _tc_add(x, pos):
    return pl.pallas_call(
        _add_body,
        grid=(BATCH // _BB,),
        in_specs=[
            pl.BlockSpec((NUM_EMB, DIM), lambda b: (0, 0)),
            pl.BlockSpec((_BB, NUM_EMB, DIM), lambda b: (b, 0, 0)),
        ],
        out_specs=pl.BlockSpec((_BB, NUM_EMB, DIM), lambda b: (b, 0, 0)),
        out_shape=jax.ShapeDtypeStruct((BATCH, NUM_EMB, DIM), jnp.float32),
        compiler_params=pltpu.CompilerParams(
            dimension_semantics=("arbitrary",),
        ),
    )(pos, x)


def kernel(x, pos_table, emb_indices):
    return _tc_add(x, pos_table)
